# baseline (speedup 1.0000x reference)
import functools

import jax
import jax.numpy as jnp
import numpy as np
from jax import lax
from jax.experimental import pallas as pl
from jax.experimental.pallas import tpu as pltpu
from jax.experimental.pallas import tpu_sc as plsc

_INTERPRET = False

HD = 64
_PAR = pltpu.CompilerParams(dimension_semantics=("parallel",))


def _ln_f32(x, s, b):
    m = jnp.mean(x, axis=-1, keepdims=True)
    v = jnp.mean((x - m) ** 2, axis=-1, keepdims=True)
    return (x - m) * lax.rsqrt(v + 1e-5) * s + b


def _gather_body(idx_ref, emb_ref, out_ref):
    out_ref[...] = emb_ref[...]


def _granule_body(idx_ref, *refs):
    out_ref = refs[-1]
    i = pl.program_id(0)
    for j in range(8):
        g = refs[j][0]
        r = idx_ref[8 * i + j] & 7
        mask = (lax.broadcasted_iota(jnp.int32, (8, 1), 0) == r).astype(g.dtype)
        out_ref[0, j, :] = jnp.sum(g * mask, axis=0)


def _gather_dma_body(idx_ref, emb_ref, out_ref, buf_ref, sem, *, BT):
    p = pl.program_id(0)
    base = p * BT

    def issue(t, c):
        g = idx_ref[base + t] >> 3
        pltpu.make_async_copy(emb_ref.at[pl.ds(g * 8, 8), :],
                              buf_ref.at[t], sem).start()
        return c

    lax.fori_loop(0, BT, issue, 0)
    sel = lax.broadcasted_iota(jnp.int32, (8, 1), 0)

    def drain(t, c):
        pltpu.make_async_copy(emb_ref.at[pl.ds(0, 8), :],
                              buf_ref.at[t], sem).wait()
        r = idx_ref[base + t] & 7
        g = buf_ref[t]
        mask = (sel == r).astype(g.dtype)
        out_ref[pl.ds(t, 1), :] = jnp.sum(g * mask, axis=0, keepdims=True)
        return c

    lax.fori_loop(0, BT, drain, 0)


def _gather_dma(emb, idx, S, D):
    BT = 256
    NP = S // BT
    grid_spec = pltpu.PrefetchScalarGridSpec(
        num_scalar_prefetch=1,
        grid=(NP,),
        in_specs=[pl.BlockSpec(memory_space=pl.ANY)],
        out_specs=pl.BlockSpec((BT, D), lambda i, idx_ref: (i, 0)),
        scratch_shapes=[
            pltpu.VMEM((BT, 8, D), jnp.float32),
            pltpu.SemaphoreType.DMA,
        ],
    )
    return pl.pallas_call(
        functools.partial(_gather_dma_body, BT=BT),
        grid_spec=grid_spec,
        out_shape=jax.ShapeDtypeStruct((S, D), jnp.float32),
        interpret=_INTERPRET,
    )(idx, emb)


def _gather_tc_granule(emb, idx, S, D):
    V = emb.shape[0]
    emb3 = emb.reshape(V // 8, 8, D)
    NT = S // 8

    def gmap(j):
        return lambda i, idx_ref: (idx_ref[8 * i + j] >> 3, 0, 0)

    grid_spec = pltpu.PrefetchScalarGridSpec(
        num_scalar_prefetch=1,
        grid=(NT,),
        in_specs=[pl.BlockSpec((1, 8, D), gmap(j)) for j in range(8)],
        out_specs=pl.BlockSpec((1, 8, D), lambda i, idx_ref: (i, 0, 0)),
    )
    out = pl.pallas_call(
        _granule_body,
        grid_spec=grid_spec,
        out_shape=jax.ShapeDtypeStruct((NT, 8, D), jnp.float32),
        interpret=_INTERPRET,
    )(idx, *([emb3] * 8))
    return out.reshape(S, D)


def _gather_tc(emb, idx, S, D):
    V = emb.shape[0]
    emb3 = emb.reshape(V, 1, D)
    grid_spec = pltpu.PrefetchScalarGridSpec(
        num_scalar_prefetch=1,
        grid=(S,),
        in_specs=[pl.BlockSpec((1, 1, D), lambda i, idx_ref: (idx_ref[i], 0, 0))],
        out_specs=pl.BlockSpec((1, 1, D), lambda i, idx_ref: (i, 0, 0)),
    )
    out = pl.pallas_call(
        _gather_body,
        grid_spec=grid_spec,
        out_shape=jax.ShapeDtypeStruct((S, 1, D), jnp.float32),
        interpret=_INTERPRET,
    )(idx, emb3)
    return out.reshape(S, D)


def _gather_sc(emb, idx, S, D):
    info = plsc.get_sparse_core_info()
    NC, NS = info.num_cores, info.num_subcores
    NW = NC * NS
    b_per_w = S // NW
    mesh = plsc.VectorSubcoreMesh(core_axis_name="c", subcore_axis_name="s")

    @functools.partial(
        pl.kernel, mesh=mesh,
        out_type=jax.ShapeDtypeStruct((S, D), jnp.float32),
        scratch_types=[
            pltpu.VMEM((b_per_w,), jnp.int32),
            pltpu.VMEM((b_per_w, D), jnp.float32),
            pltpu.SemaphoreType.DMA,
        ],
        compiler_params=pltpu.CompilerParams(use_tc_tiling_on_sc=True),
    )
    def sc_gather(table_hbm, idx_hbm, out_hbm, idx_v, rows_v, sem):
        wid = lax.axis_index("s") * NC + lax.axis_index("c")
        base = wid * b_per_w
        pltpu.sync_copy(idx_hbm.at[pl.ds(base, b_per_w)], idx_v)
        pltpu.async_copy(table_hbm.at[idx_v], rows_v, sem).wait()
        pltpu.sync_copy(rows_v, out_hbm.at[pl.ds(base, b_per_w)])

    return sc_gather(emb, idx)


def _qkv_body(h_ref, s_ref, b_ref, w_ref, bqkv_ref, qkv_ref):
    hn = _ln_f32(h_ref[...], s_ref[...], b_ref[...])
    acc = jnp.dot(hn.astype(jnp.bfloat16), w_ref[...],
                  preferred_element_type=jnp.float32)
    qkv_ref[...] = (acc + bqkv_ref[...]).astype(jnp.bfloat16)


def _attn_body(q_ref, k_ref, v_ref, o_ref, *, BQ, S, H):
    i = pl.program_id(0)
    scale = 1.0 / np.sqrt(HD)
    rows = lax.broadcasted_iota(jnp.int32, (BQ, BQ), 0)
    cols = lax.broadcasted_iota(jnp.int32, (BQ, BQ), 1)
    diag_mask = rows >= cols
    for h in range(H):
        q = q_ref[:, h * HD:(h + 1) * HD]

        def body(j, carry):
            m, l, acc = carry
            kj = k_ref[pl.ds(j * BQ, BQ), h * HD:(h + 1) * HD]
            vj = v_ref[pl.ds(j * BQ, BQ), h * HD:(h + 1) * HD]
            sc = lax.dot_general(q, kj, (((1,), (1,)), ((), ())),
                                 preferred_element_type=jnp.float32) * scale
            sc = jnp.where(jnp.logical_or(j < i, diag_mask), sc, -1e9)
            m_new = jnp.maximum(m, jnp.max(sc, axis=-1, keepdims=True))
            alpha = jnp.exp(m - m_new)
            p = jnp.exp(sc - m_new)
            l = l * alpha + jnp.sum(p, axis=-1, keepdims=True)
            acc = acc * alpha + jnp.dot(p.astype(jnp.bfloat16), vj,
                                        preferred_element_type=jnp.float32)
            return m_new, l, acc

        m0 = jnp.full((BQ, 1), -1e30, jnp.float32)
        l0 = jnp.zeros((BQ, 1), jnp.float32)
        a0 = jnp.zeros((BQ, HD), jnp.float32)
        _, l, acc = lax.fori_loop(0, i + 1, body, (m0, l0, a0))
        o = acc * (1.0 / l)
        o_ref[:, h * HD:(h + 1) * HD] = o.astype(jnp.bfloat16)


def _post_body(o_ref, h_ref, wo_ref, bo_ref, s2_ref, b2ln_ref,
               w1_ref, b1_ref, w2_ref, b2_ref, out_ref, out_bf_ref):
    h = h_ref[...] + jnp.dot(o_ref[...], wo_ref[...],
                             preferred_element_type=jnp.float32) + bo_ref[...]
    hn2 = _ln_f32(h, s2_ref[...], b2ln_ref[...])
    ff = jnp.dot(hn2.astype(jnp.bfloat16), w1_ref[...],
                 preferred_element_type=jnp.float32) + b1_ref[...]
    ff = jax.nn.gelu(ff)
    out = h + jnp.dot(ff.astype(jnp.bfloat16), w2_ref[...],
                      preferred_element_type=jnp.float32) + b2_ref[...]
    out_ref[...] = out
    out_bf_ref[...] = out.astype(jnp.bfloat16)


def _head_body(h_ref, w_ref, o_ref):
    o_ref[0] = lax.dot_general(h_ref[...], w_ref[...].astype(jnp.bfloat16),
                               (((1,), (1,)), ((), ())),
                               preferred_element_type=jnp.float32)


def _layer(h, s1, b1ln, wqkv_bf, bqkv, wo_bf, bo, s2, b2ln,
           w1_bf, b1, w2_bf, b2, S, D, H, BS):
    NB = S // BS
    F = w1_bf.shape[1]
    qkv = pl.pallas_call(
        _qkv_body,
        grid=(NB,),
        in_specs=[
            pl.BlockSpec((BS, D), lambda i: (i, 0)),
            pl.BlockSpec((1, D), lambda i: (0, 0)),
            pl.BlockSpec((1, D), lambda i: (0, 0)),
            pl.BlockSpec((D, 3 * D), lambda i: (0, 0)),
            pl.BlockSpec((1, 3 * D), lambda i: (0, 0)),
        ],
        out_specs=pl.BlockSpec((BS, 3 * D), lambda i: (i, 0)),
        out_shape=jax.ShapeDtypeStruct((S, 3 * D), jnp.bfloat16),
        compiler_params=_PAR,
        interpret=_INTERPRET,
    )(h, s1, b1ln, wqkv_bf, bqkv)

    o = pl.pallas_call(
        functools.partial(_attn_body, BQ=BS, S=S, H=H),
        grid=(NB,),
        in_specs=[
            pl.BlockSpec((BS, D), lambda i: (i, 0)),
            pl.BlockSpec((S, D), lambda i: (0, 1)),
            pl.BlockSpec((S, D), lambda i: (0, 2)),
        ],
        out_specs=pl.BlockSpec((BS, D), lambda i: (i, 0)),
        out_shape=jax.ShapeDtypeStruct((S, D), jnp.bfloat16),
        compiler_params=_PAR,
        interpret=_INTERPRET,
    )(qkv, qkv, qkv)

    h = pl.pallas_call(
        _post_body,
        grid=(NB,),
        in_specs=[
            pl.BlockSpec((BS, D), lambda i: (i, 0)),
            pl.BlockSpec((BS, D), lambda i: (i, 0)),
            pl.BlockSpec((D, D), lambda i: (0, 0)),
            pl.BlockSpec((1, D), lambda i: (0, 0)),
            pl.BlockSpec((1, D), lambda i: (0, 0)),
            pl.BlockSpec((1, D), lambda i: (0, 0)),
            pl.BlockSpec((D, F), lambda i: (0, 0)),
            pl.BlockSpec((1, F), lambda i: (0, 0)),
            pl.BlockSpec((F, D), lambda i: (0, 0)),
            pl.BlockSpec((1, D), lambda i: (0, 0)),
        ],
        out_specs=[pl.BlockSpec((BS, D), lambda i: (i, 0)),
                   pl.BlockSpec((BS, D), lambda i: (i, 0))],
        out_shape=[jax.ShapeDtypeStruct((S, D), jnp.float32),
                   jax.ShapeDtypeStruct((S, D), jnp.bfloat16)],
        compiler_params=_PAR,
        interpret=_INTERPRET,
    )(o, h, wo_bf, bo, s2, b2ln, w1_bf, b1, w2_bf, b2)
    return h[0], h[1]


def kernel(x, emb, ln1_s, ln1_b, wqkv, bqkv, wo, bo, ln2_s, ln2_b,
           w1, b1, w2, b2, w_out):
    B, S = x.shape
    V, D = emb.shape
    L = wqkv.shape[0]
    H = D // HD
    BS = 256
    VB = 1024

    idx = x.reshape(S).astype(jnp.int32)
    h = _gather_sc(emb, idx, S, D)

    bf = jnp.bfloat16
    h_bf = None
    for l in range(L):
        h, h_bf = _layer(
            h,
            ln1_s[l].reshape(1, D), ln1_b[l].reshape(1, D),
            wqkv[l].astype(bf), bqkv[l].reshape(1, 3 * D),
            wo[l].astype(bf), bo[l].reshape(1, D),
            ln2_s[l].reshape(1, D), ln2_b[l].reshape(1, D),
            w1[l].astype(bf), b1[l].reshape(1, -1),
            w2[l].astype(bf), b2[l].reshape(1, D),
            S, D, H, BS,
        )

    NV = (V + VB - 1) // VB
    logits = pl.pallas_call(
        _head_body,
        grid=(NV,),
        in_specs=[
            pl.BlockSpec((S, D), lambda j: (0, 0)),
            pl.BlockSpec((VB, D), lambda j: (j, 0)),
        ],
        out_specs=pl.BlockSpec((1, S, VB), lambda j: (0, 0, j)),
        out_shape=jax.ShapeDtypeStruct((B, S, V), jnp.float32),
        compiler_params=_PAR,
        interpret=_INTERPRET,
    )(h_bf, w_out)
    return logits


# speedup vs baseline: 1.0026x; 1.0026x over previous
import functools

import jax
import jax.numpy as jnp
import numpy as np
from jax import lax
from jax.experimental import pallas as pl
from jax.experimental.pallas import tpu as pltpu
from jax.experimental.pallas import tpu_sc as plsc

_INTERPRET = False

HD = 64
_PAR = pltpu.CompilerParams(dimension_semantics=("arbitrary",))


def _ln_f32(x, s, b):
    m = jnp.mean(x, axis=-1, keepdims=True)
    v = jnp.mean((x - m) ** 2, axis=-1, keepdims=True)
    return (x - m) * lax.rsqrt(v + 1e-5) * s + b


def _gather_body(idx_ref, emb_ref, out_ref):
    out_ref[...] = emb_ref[...]


def _granule_body(idx_ref, *refs):
    out_ref = refs[-1]
    i = pl.program_id(0)
    for j in range(8):
        g = refs[j][0]
        r = idx_ref[8 * i + j] & 7
        mask = (lax.broadcasted_iota(jnp.int32, (8, 1), 0) == r).astype(g.dtype)
        out_ref[0, j, :] = jnp.sum(g * mask, axis=0)


def _gather_dma_body(idx_ref, emb_ref, out_ref, buf_ref, sem, *, BT):
    p = pl.program_id(0)
    base = p * BT

    def issue(t, c):
        g = idx_ref[base + t] >> 3
        pltpu.make_async_copy(emb_ref.at[pl.ds(g * 8, 8), :],
                              buf_ref.at[t], sem).start()
        return c

    lax.fori_loop(0, BT, issue, 0)
    sel = lax.broadcasted_iota(jnp.int32, (8, 1), 0)

    def drain(t, c):
        pltpu.make_async_copy(emb_ref.at[pl.ds(0, 8), :],
                              buf_ref.at[t], sem).wait()
        r = idx_ref[base + t] & 7
        g = buf_ref[t]
        mask = (sel == r).astype(g.dtype)
        out_ref[pl.ds(t, 1), :] = jnp.sum(g * mask, axis=0, keepdims=True)
        return c

    lax.fori_loop(0, BT, drain, 0)


def _gather_dma(emb, idx, S, D):
    BT = 256
    NP = S // BT
    grid_spec = pltpu.PrefetchScalarGridSpec(
        num_scalar_prefetch=1,
        grid=(NP,),
        in_specs=[pl.BlockSpec(memory_space=pl.ANY)],
        out_specs=pl.BlockSpec((BT, D), lambda i, idx_ref: (i, 0)),
        scratch_shapes=[
            pltpu.VMEM((BT, 8, D), jnp.float32),
            pltpu.SemaphoreType.DMA,
        ],
    )
    return pl.pallas_call(
        functools.partial(_gather_dma_body, BT=BT),
        grid_spec=grid_spec,
        out_shape=jax.ShapeDtypeStruct((S, D), jnp.float32),
        interpret=_INTERPRET,
    )(idx, emb)


def _gather_tc_granule(emb, idx, S, D):
    V = emb.shape[0]
    emb3 = emb.reshape(V // 8, 8, D)
    NT = S // 8

    def gmap(j):
        return lambda i, idx_ref: (idx_ref[8 * i + j] >> 3, 0, 0)

    grid_spec = pltpu.PrefetchScalarGridSpec(
        num_scalar_prefetch=1,
        grid=(NT,),
        in_specs=[pl.BlockSpec((1, 8, D), gmap(j)) for j in range(8)],
        out_specs=pl.BlockSpec((1, 8, D), lambda i, idx_ref: (i, 0, 0)),
    )
    out = pl.pallas_call(
        _granule_body,
        grid_spec=grid_spec,
        out_shape=jax.ShapeDtypeStruct((NT, 8, D), jnp.float32),
        interpret=_INTERPRET,
    )(idx, *([emb3] * 8))
    return out.reshape(S, D)


def _gather_tc(emb, idx, S, D):
    V = emb.shape[0]
    emb3 = emb.reshape(V, 1, D)
    grid_spec = pltpu.PrefetchScalarGridSpec(
        num_scalar_prefetch=1,
        grid=(S,),
        in_specs=[pl.BlockSpec((1, 1, D), lambda i, idx_ref: (idx_ref[i], 0, 0))],
        out_specs=pl.BlockSpec((1, 1, D), lambda i, idx_ref: (i, 0, 0)),
    )
    out = pl.pallas_call(
        _gather_body,
        grid_spec=grid_spec,
        out_shape=jax.ShapeDtypeStruct((S, 1, D), jnp.float32),
        interpret=_INTERPRET,
    )(idx, emb3)
    return out.reshape(S, D)


def _gather_sc(emb, idx, S, D):
    info = plsc.get_sparse_core_info()
    NC, NS = info.num_cores, info.num_subcores
    NW = NC * NS
    b_per_w = S // NW
    mesh = plsc.VectorSubcoreMesh(core_axis_name="c", subcore_axis_name="s")

    @functools.partial(
        pl.kernel, mesh=mesh,
        out_type=jax.ShapeDtypeStruct((S, D), jnp.float32),
        scratch_types=[
            pltpu.VMEM((b_per_w,), jnp.int32),
            pltpu.VMEM((b_per_w, D), jnp.float32),
            pltpu.SemaphoreType.DMA,
        ],
        compiler_params=pltpu.CompilerParams(use_tc_tiling_on_sc=True),
    )
    def sc_gather(table_hbm, idx_hbm, out_hbm, idx_v, rows_v, sem):
        wid = lax.axis_index("s") * NC + lax.axis_index("c")
        base = wid * b_per_w
        pltpu.sync_copy(idx_hbm.at[pl.ds(base, b_per_w)], idx_v)
        pltpu.async_copy(table_hbm.at[idx_v], rows_v, sem).wait()
        pltpu.sync_copy(rows_v, out_hbm.at[pl.ds(base, b_per_w)])

    return sc_gather(emb, idx)


def _qkv_body(h_ref, s_ref, b_ref, w_ref, bqkv_ref, qkv_ref):
    hn = _ln_f32(h_ref[...], s_ref[...], b_ref[...])
    acc = jnp.dot(hn.astype(jnp.bfloat16), w_ref[...],
                  preferred_element_type=jnp.float32)
    qkv_ref[...] = (acc + bqkv_ref[...]).astype(jnp.bfloat16)


def _attn_body(q_ref, k_ref, v_ref, o_ref, *, BQ, S, H):
    i = pl.program_id(0)
    scale = 1.0 / np.sqrt(HD)
    rows = lax.broadcasted_iota(jnp.int32, (BQ, BQ), 0)
    cols = lax.broadcasted_iota(jnp.int32, (BQ, BQ), 1)
    diag_mask = rows >= cols
    for h in range(H):
        q = q_ref[:, h * HD:(h + 1) * HD]

        def body(j, carry):
            m, l, acc = carry
            kj = k_ref[pl.ds(j * BQ, BQ), h * HD:(h + 1) * HD]
            vj = v_ref[pl.ds(j * BQ, BQ), h * HD:(h + 1) * HD]
            sc = lax.dot_general(q, kj, (((1,), (1,)), ((), ())),
                                 preferred_element_type=jnp.float32) * scale
            sc = jnp.where(jnp.logical_or(j < i, diag_mask), sc, -1e9)
            m_new = jnp.maximum(m, jnp.max(sc, axis=-1, keepdims=True))
            alpha = jnp.exp(m - m_new)
            p = jnp.exp(sc - m_new)
            l = l * alpha + jnp.sum(p, axis=-1, keepdims=True)
            acc = acc * alpha + jnp.dot(p.astype(jnp.bfloat16), vj,
                                        preferred_element_type=jnp.float32)
            return m_new, l, acc

        m0 = jnp.full((BQ, 1), -1e30, jnp.float32)
        l0 = jnp.zeros((BQ, 1), jnp.float32)
        a0 = jnp.zeros((BQ, HD), jnp.float32)
        _, l, acc = lax.fori_loop(0, i + 1, body, (m0, l0, a0))
        o = acc * (1.0 / l)
        o_ref[:, h * HD:(h + 1) * HD] = o.astype(jnp.bfloat16)


def _post_body(o_ref, h_ref, wo_ref, bo_ref, s2_ref, b2ln_ref,
               w1_ref, b1_ref, w2_ref, b2_ref, out_ref, out_bf_ref):
    h = h_ref[...] + jnp.dot(o_ref[...], wo_ref[...],
                             preferred_element_type=jnp.float32) + bo_ref[...]
    hn2 = _ln_f32(h, s2_ref[...], b2ln_ref[...])
    ff = jnp.dot(hn2.astype(jnp.bfloat16), w1_ref[...],
                 preferred_element_type=jnp.float32) + b1_ref[...]
    ff = jax.nn.gelu(ff)
    out = h + jnp.dot(ff.astype(jnp.bfloat16), w2_ref[...],
                      preferred_element_type=jnp.float32) + b2_ref[...]
    out_ref[...] = out
    out_bf_ref[...] = out.astype(jnp.bfloat16)


def _head_body(h_ref, w_ref, o_ref):
    o_ref[0] = lax.dot_general(h_ref[...], w_ref[...].astype(jnp.bfloat16),
                               (((1,), (1,)), ((), ())),
                               preferred_element_type=jnp.float32)


def _layer(h, s1, b1ln, wqkv_bf, bqkv, wo_bf, bo, s2, b2ln,
           w1_bf, b1, w2_bf, b2, S, D, H, BS):
    NB = S // BS
    F = w1_bf.shape[1]
    qkv = pl.pallas_call(
        _qkv_body,
        grid=(NB,),
        in_specs=[
            pl.BlockSpec((BS, D), lambda i: (i, 0)),
            pl.BlockSpec((1, D), lambda i: (0, 0)),
            pl.BlockSpec((1, D), lambda i: (0, 0)),
            pl.BlockSpec((D, 3 * D), lambda i: (0, 0)),
            pl.BlockSpec((1, 3 * D), lambda i: (0, 0)),
        ],
        out_specs=pl.BlockSpec((BS, 3 * D), lambda i: (i, 0)),
        out_shape=jax.ShapeDtypeStruct((S, 3 * D), jnp.bfloat16),
        compiler_params=_PAR,
        interpret=_INTERPRET,
    )(h, s1, b1ln, wqkv_bf, bqkv)

    o = pl.pallas_call(
        functools.partial(_attn_body, BQ=BS, S=S, H=H),
        grid=(NB,),
        in_specs=[
            pl.BlockSpec((BS, D), lambda i: (i, 0)),
            pl.BlockSpec((S, D), lambda i: (0, 1)),
            pl.BlockSpec((S, D), lambda i: (0, 2)),
        ],
        out_specs=pl.BlockSpec((BS, D), lambda i: (i, 0)),
        out_shape=jax.ShapeDtypeStruct((S, D), jnp.bfloat16),
        compiler_params=_PAR,
        interpret=_INTERPRET,
    )(qkv, qkv, qkv)

    h = pl.pallas_call(
        _post_body,
        grid=(NB,),
        in_specs=[
            pl.BlockSpec((BS, D), lambda i: (i, 0)),
            pl.BlockSpec((BS, D), lambda i: (i, 0)),
            pl.BlockSpec((D, D), lambda i: (0, 0)),
            pl.BlockSpec((1, D), lambda i: (0, 0)),
            pl.BlockSpec((1, D), lambda i: (0, 0)),
            pl.BlockSpec((1, D), lambda i: (0, 0)),
            pl.BlockSpec((D, F), lambda i: (0, 0)),
            pl.BlockSpec((1, F), lambda i: (0, 0)),
            pl.BlockSpec((F, D), lambda i: (0, 0)),
            pl.BlockSpec((1, D), lambda i: (0, 0)),
        ],
        out_specs=[pl.BlockSpec((BS, D), lambda i: (i, 0)),
                   pl.BlockSpec((BS, D), lambda i: (i, 0))],
        out_shape=[jax.ShapeDtypeStruct((S, D), jnp.float32),
                   jax.ShapeDtypeStruct((S, D), jnp.bfloat16)],
        compiler_params=_PAR,
        interpret=_INTERPRET,
    )(o, h, wo_bf, bo, s2, b2ln, w1_bf, b1, w2_bf, b2)
    return h[0], h[1]


def kernel(x, emb, ln1_s, ln1_b, wqkv, bqkv, wo, bo, ln2_s, ln2_b,
           w1, b1, w2, b2, w_out):
    B, S = x.shape
    V, D = emb.shape
    L = wqkv.shape[0]
    H = D // HD
    BS = 256
    VB = 1024

    idx = x.reshape(S).astype(jnp.int32)
    h = _gather_sc(emb, idx, S, D)

    bf = jnp.bfloat16
    h_bf = None
    for l in range(L):
        h, h_bf = _layer(
            h,
            ln1_s[l].reshape(1, D), ln1_b[l].reshape(1, D),
            wqkv[l].astype(bf), bqkv[l].reshape(1, 3 * D),
            wo[l].astype(bf), bo[l].reshape(1, D),
            ln2_s[l].reshape(1, D), ln2_b[l].reshape(1, D),
            w1[l].astype(bf), b1[l].reshape(1, -1),
            w2[l].astype(bf), b2[l].reshape(1, D),
            S, D, H, BS,
        )

    NV = (V + VB - 1) // VB
    logits = pl.pallas_call(
        _head_body,
        grid=(NV,),
        in_specs=[
            pl.BlockSpec((S, D), lambda j: (0, 0)),
            pl.BlockSpec((VB, D), lambda j: (j, 0)),
        ],
        out_specs=pl.BlockSpec((1, S, VB), lambda j: (0, 0, j)),
        out_shape=jax.ShapeDtypeStruct((B, S, V), jnp.float32),
        compiler_params=_PAR,
        interpret=_INTERPRET,
    )(h_bf, w_out)
    return logits


# speedup vs baseline: 1.6816x; 1.6772x over previous
import functools

import jax
import jax.numpy as jnp
import numpy as np
from jax import lax
from jax.experimental import pallas as pl
from jax.experimental.pallas import tpu as pltpu
from jax.experimental.pallas import tpu_sc as plsc

_INTERPRET = False

HD = 64
_PAR = pltpu.CompilerParams(dimension_semantics=("arbitrary",))


def _ln_f32(x, s, b):
    m = jnp.mean(x, axis=-1, keepdims=True)
    v = jnp.mean((x - m) ** 2, axis=-1, keepdims=True)
    return (x - m) * lax.rsqrt(v + 1e-5) * s + b


def _gather_body(idx_ref, emb_ref, out_ref):
    out_ref[...] = emb_ref[...]


def _granule_body(idx_ref, *refs):
    out_ref = refs[-1]
    i = pl.program_id(0)
    for j in range(8):
        g = refs[j][0]
        r = idx_ref[8 * i + j] & 7
        mask = (lax.broadcasted_iota(jnp.int32, (8, 1), 0) == r).astype(g.dtype)
        out_ref[0, j, :] = jnp.sum(g * mask, axis=0)


def _gather_dma_body(idx_ref, emb_ref, out_ref, buf_ref, sem, *, BT):
    p = pl.program_id(0)
    base = p * BT

    def issue(t, c):
        g = idx_ref[base + t] >> 3
        pltpu.make_async_copy(emb_ref.at[pl.ds(g * 8, 8), :],
                              buf_ref.at[t], sem).start()
        return c

    lax.fori_loop(0, BT, issue, 0)
    sel = lax.broadcasted_iota(jnp.int32, (8, 1), 0)

    def drain(t, c):
        pltpu.make_async_copy(emb_ref.at[pl.ds(0, 8), :],
                              buf_ref.at[t], sem).wait()
        r = idx_ref[base + t] & 7
        g = buf_ref[t]
        mask = (sel == r).astype(g.dtype)
        out_ref[pl.ds(t, 1), :] = jnp.sum(g * mask, axis=0, keepdims=True)
        return c

    lax.fori_loop(0, BT, drain, 0)


def _gather_dma(emb, idx, S, D):
    BT = 256
    NP = S // BT
    grid_spec = pltpu.PrefetchScalarGridSpec(
        num_scalar_prefetch=1,
        grid=(NP,),
        in_specs=[pl.BlockSpec(memory_space=pl.ANY)],
        out_specs=pl.BlockSpec((BT, D), lambda i, idx_ref: (i, 0)),
        scratch_shapes=[
            pltpu.VMEM((BT, 8, D), jnp.float32),
            pltpu.SemaphoreType.DMA,
        ],
    )
    return pl.pallas_call(
        functools.partial(_gather_dma_body, BT=BT),
        grid_spec=grid_spec,
        out_shape=jax.ShapeDtypeStruct((S, D), jnp.float32),
        interpret=_INTERPRET,
    )(idx, emb)


def _gather_tc_granule(emb, idx, S, D):
    V = emb.shape[0]
    emb3 = emb.reshape(V // 8, 8, D)
    NT = S // 8

    def gmap(j):
        return lambda i, idx_ref: (idx_ref[8 * i + j] >> 3, 0, 0)

    grid_spec = pltpu.PrefetchScalarGridSpec(
        num_scalar_prefetch=1,
        grid=(NT,),
        in_specs=[pl.BlockSpec((1, 8, D), gmap(j)) for j in range(8)],
        out_specs=pl.BlockSpec((1, 8, D), lambda i, idx_ref: (i, 0, 0)),
    )
    out = pl.pallas_call(
        _granule_body,
        grid_spec=grid_spec,
        out_shape=jax.ShapeDtypeStruct((NT, 8, D), jnp.float32),
        interpret=_INTERPRET,
    )(idx, *([emb3] * 8))
    return out.reshape(S, D)


def _gather_tc(emb, idx, S, D):
    V = emb.shape[0]
    emb3 = emb.reshape(V, 1, D)
    grid_spec = pltpu.PrefetchScalarGridSpec(
        num_scalar_prefetch=1,
        grid=(S,),
        in_specs=[pl.BlockSpec((1, 1, D), lambda i, idx_ref: (idx_ref[i], 0, 0))],
        out_specs=pl.BlockSpec((1, 1, D), lambda i, idx_ref: (i, 0, 0)),
    )
    out = pl.pallas_call(
        _gather_body,
        grid_spec=grid_spec,
        out_shape=jax.ShapeDtypeStruct((S, 1, D), jnp.float32),
        interpret=_INTERPRET,
    )(idx, emb3)
    return out.reshape(S, D)


def _gather_sc(emb, idx, S, D):
    info = plsc.get_sparse_core_info()
    NC, NS = info.num_cores, info.num_subcores
    NW = NC * NS
    b_per_w = S // NW
    mesh = plsc.VectorSubcoreMesh(core_axis_name="c", subcore_axis_name="s")

    @functools.partial(
        pl.kernel, mesh=mesh,
        out_type=jax.ShapeDtypeStruct((S, D), jnp.float32),
        scratch_types=[
            pltpu.VMEM((b_per_w,), jnp.int32),
            pltpu.VMEM((b_per_w, D), jnp.float32),
            pltpu.SemaphoreType.DMA,
        ],
        compiler_params=pltpu.CompilerParams(use_tc_tiling_on_sc=True),
    )
    def sc_gather(table_hbm, idx_hbm, out_hbm, idx_v, rows_v, sem):
        wid = lax.axis_index("s") * NC + lax.axis_index("c")
        base = wid * b_per_w
        pltpu.sync_copy(idx_hbm.at[pl.ds(base, b_per_w)], idx_v)
        pltpu.async_copy(table_hbm.at[idx_v], rows_v, sem).wait()
        pltpu.sync_copy(rows_v, out_hbm.at[pl.ds(base, b_per_w)])

    return sc_gather(emb, idx)


def _qkv_body(h_ref, s_ref, b_ref, w_ref, bqkv_ref, qkv_ref):
    hn = _ln_f32(h_ref[...], s_ref[...], b_ref[...])
    acc = jnp.dot(hn.astype(jnp.bfloat16), w_ref[...],
                  preferred_element_type=jnp.float32)
    qkv_ref[...] = (acc + bqkv_ref[...]).astype(jnp.bfloat16)


def _attn_body(q_ref, k_ref, v_ref, o_ref, *, BQ, S, H):
    i = pl.program_id(0)
    scale = 1.0 / np.sqrt(HD)
    rows = lax.broadcasted_iota(jnp.int32, (BQ, BQ), 0)
    cols = lax.broadcasted_iota(jnp.int32, (BQ, BQ), 1)
    diag_mask = rows >= cols
    for h in range(H):
        q = q_ref[:, h * HD:(h + 1) * HD]

        def body(j, carry):
            m, l, acc = carry
            kj = k_ref[pl.ds(j * BQ, BQ), h * HD:(h + 1) * HD]
            vj = v_ref[pl.ds(j * BQ, BQ), h * HD:(h + 1) * HD]
            sc = lax.dot_general(q, kj, (((1,), (1,)), ((), ())),
                                 preferred_element_type=jnp.float32) * scale
            sc = jnp.where(jnp.logical_or(j < i, diag_mask), sc, -1e9)
            m_new = jnp.maximum(m, jnp.max(sc, axis=-1, keepdims=True))
            alpha = jnp.exp(m - m_new)
            p = jnp.exp(sc - m_new)
            l = l * alpha + jnp.sum(p, axis=-1, keepdims=True)
            acc = acc * alpha + jnp.dot(p.astype(jnp.bfloat16), vj,
                                        preferred_element_type=jnp.float32)
            return m_new, l, acc

        m0 = jnp.full((BQ, 1), -1e30, jnp.float32)
        l0 = jnp.zeros((BQ, 1), jnp.float32)
        a0 = jnp.zeros((BQ, HD), jnp.float32)
        _, l, acc = lax.fori_loop(0, i + 1, body, (m0, l0, a0))
        o = acc * (1.0 / l)
        o_ref[:, h * HD:(h + 1) * HD] = o.astype(jnp.bfloat16)


def _post_body(o_ref, h_ref, wo_ref, bo_ref, s2_ref, b2ln_ref,
               w1_ref, b1_ref, w2_ref, b2_ref, out_ref, out_bf_ref):
    h = h_ref[...] + jnp.dot(o_ref[...], wo_ref[...],
                             preferred_element_type=jnp.float32) + bo_ref[...]
    hn2 = _ln_f32(h, s2_ref[...], b2ln_ref[...])
    ff = jnp.dot(hn2.astype(jnp.bfloat16), w1_ref[...],
                 preferred_element_type=jnp.float32) + b1_ref[...]
    ff = jax.nn.gelu(ff)
    out = h + jnp.dot(ff.astype(jnp.bfloat16), w2_ref[...],
                      preferred_element_type=jnp.float32) + b2_ref[...]
    out_ref[...] = out
    out_bf_ref[...] = out.astype(jnp.bfloat16)


def _head_body(h_ref, w_ref, o_ref):
    o_ref[...] = lax.dot_general(h_ref[...], w_ref[...].astype(jnp.bfloat16),
                                 (((1,), (1,)), ((), ())),
                                 preferred_element_type=jnp.float32)


def _layer(h, s1, b1ln, wqkv_bf, bqkv, wo_bf, bo, s2, b2ln,
           w1_bf, b1, w2_bf, b2, S, D, H, BS):
    NB = S // BS
    F = w1_bf.shape[1]
    qkv = pl.pallas_call(
        _qkv_body,
        grid=(NB,),
        in_specs=[
            pl.BlockSpec((BS, D), lambda i: (i, 0)),
            pl.BlockSpec((1, D), lambda i: (0, 0)),
            pl.BlockSpec((1, D), lambda i: (0, 0)),
            pl.BlockSpec((D, 3 * D), lambda i: (0, 0)),
            pl.BlockSpec((1, 3 * D), lambda i: (0, 0)),
        ],
        out_specs=pl.BlockSpec((BS, 3 * D), lambda i: (i, 0)),
        out_shape=jax.ShapeDtypeStruct((S, 3 * D), jnp.bfloat16),
        compiler_params=_PAR,
        interpret=_INTERPRET,
    )(h, s1, b1ln, wqkv_bf, bqkv)

    o = pl.pallas_call(
        functools.partial(_attn_body, BQ=BS, S=S, H=H),
        grid=(NB,),
        in_specs=[
            pl.BlockSpec((BS, D), lambda i: (i, 0)),
            pl.BlockSpec((S, D), lambda i: (0, 1)),
            pl.BlockSpec((S, D), lambda i: (0, 2)),
        ],
        out_specs=pl.BlockSpec((BS, D), lambda i: (i, 0)),
        out_shape=jax.ShapeDtypeStruct((S, D), jnp.bfloat16),
        compiler_params=_PAR,
        interpret=_INTERPRET,
    )(qkv, qkv, qkv)

    h = pl.pallas_call(
        _post_body,
        grid=(NB,),
        in_specs=[
            pl.BlockSpec((BS, D), lambda i: (i, 0)),
            pl.BlockSpec((BS, D), lambda i: (i, 0)),
            pl.BlockSpec((D, D), lambda i: (0, 0)),
            pl.BlockSpec((1, D), lambda i: (0, 0)),
            pl.BlockSpec((1, D), lambda i: (0, 0)),
            pl.BlockSpec((1, D), lambda i: (0, 0)),
            pl.BlockSpec((D, F), lambda i: (0, 0)),
            pl.BlockSpec((1, F), lambda i: (0, 0)),
            pl.BlockSpec((F, D), lambda i: (0, 0)),
            pl.BlockSpec((1, D), lambda i: (0, 0)),
        ],
        out_specs=[pl.BlockSpec((BS, D), lambda i: (i, 0)),
                   pl.BlockSpec((BS, D), lambda i: (i, 0))],
        out_shape=[jax.ShapeDtypeStruct((S, D), jnp.float32),
                   jax.ShapeDtypeStruct((S, D), jnp.bfloat16)],
        compiler_params=_PAR,
        interpret=_INTERPRET,
    )(o, h, wo_bf, bo, s2, b2ln, w1_bf, b1, w2_bf, b2)
    return h[0], h[1]


def kernel(x, emb, ln1_s, ln1_b, wqkv, bqkv, wo, bo, ln2_s, ln2_b,
           w1, b1, w2, b2, w_out):
    B, S = x.shape
    V, D = emb.shape
    L = wqkv.shape[0]
    H = D // HD
    BS = 256
    VB = 1024

    idx = x.reshape(S).astype(jnp.int32)
    h = _gather_sc(emb, idx, S, D)

    bf = jnp.bfloat16
    h_bf = None
    for l in range(L):
        h, h_bf = _layer(
            h,
            ln1_s[l].reshape(1, D), ln1_b[l].reshape(1, D),
            wqkv[l].astype(bf), bqkv[l].reshape(1, 3 * D),
            wo[l].astype(bf), bo[l].reshape(1, D),
            ln2_s[l].reshape(1, D), ln2_b[l].reshape(1, D),
            w1[l].astype(bf), b1[l].reshape(1, -1),
            w2[l].astype(bf), b2[l].reshape(1, D),
            S, D, H, BS,
        )

    NV = (V + VB - 1) // VB
    logits = pl.pallas_call(
        _head_body,
        grid=(NV,),
        in_specs=[
            pl.BlockSpec((S, D), lambda j: (0, 0)),
            pl.BlockSpec((VB, D), lambda j: (j, 0)),
        ],
        out_specs=pl.BlockSpec((S, VB), lambda j: (0, j)),
        out_shape=jax.ShapeDtypeStruct((S, V), jnp.float32),
        compiler_params=_PAR,
        interpret=_INTERPRET,
    )(h_bf, w_out)
    return logits.reshape(B, S, V)


# speedup vs baseline: 1.6821x; 1.0003x over previous
import functools

import jax
import jax.numpy as jnp
import numpy as np
from jax import lax
from jax.experimental import pallas as pl
from jax.experimental.pallas import tpu as pltpu
from jax.experimental.pallas import tpu_sc as plsc

_INTERPRET = False

HD = 64
_PAR = pltpu.CompilerParams(dimension_semantics=("arbitrary",))


def _ln_f32(x, s, b):
    m = jnp.mean(x, axis=-1, keepdims=True)
    v = jnp.mean((x - m) ** 2, axis=-1, keepdims=True)
    return (x - m) * lax.rsqrt(v + 1e-5) * s + b


def _gather_body(idx_ref, emb_ref, out_ref):
    out_ref[...] = emb_ref[...]


def _granule_body(idx_ref, *refs):
    out_ref = refs[-1]
    i = pl.program_id(0)
    for j in range(8):
        g = refs[j][0]
        r = idx_ref[8 * i + j] & 7
        mask = (lax.broadcasted_iota(jnp.int32, (8, 1), 0) == r).astype(g.dtype)
        out_ref[0, j, :] = jnp.sum(g * mask, axis=0)


def _gather_dma_body(idx_ref, emb_ref, out_ref, buf_ref, sem, *, BT):
    p = pl.program_id(0)
    base = p * BT

    def issue(t, c):
        g = idx_ref[base + t] >> 3
        pltpu.make_async_copy(emb_ref.at[pl.ds(g * 8, 8), :],
                              buf_ref.at[t], sem).start()
        return c

    lax.fori_loop(0, BT, issue, 0)
    sel = lax.broadcasted_iota(jnp.int32, (8, 1), 0)

    def drain(t, c):
        pltpu.make_async_copy(emb_ref.at[pl.ds(0, 8), :],
                              buf_ref.at[t], sem).wait()
        r = idx_ref[base + t] & 7
        g = buf_ref[t]
        mask = (sel == r).astype(g.dtype)
        out_ref[pl.ds(t, 1), :] = jnp.sum(g * mask, axis=0, keepdims=True)
        return c

    lax.fori_loop(0, BT, drain, 0)


def _gather_dma(emb, idx, S, D):
    BT = 256
    NP = S // BT
    grid_spec = pltpu.PrefetchScalarGridSpec(
        num_scalar_prefetch=1,
        grid=(NP,),
        in_specs=[pl.BlockSpec(memory_space=pl.ANY)],
        out_specs=pl.BlockSpec((BT, D), lambda i, idx_ref: (i, 0)),
        scratch_shapes=[
            pltpu.VMEM((BT, 8, D), jnp.float32),
            pltpu.SemaphoreType.DMA,
        ],
    )
    return pl.pallas_call(
        functools.partial(_gather_dma_body, BT=BT),
        grid_spec=grid_spec,
        out_shape=jax.ShapeDtypeStruct((S, D), jnp.float32),
        interpret=_INTERPRET,
    )(idx, emb)


def _gather_tc_granule(emb, idx, S, D):
    V = emb.shape[0]
    emb3 = emb.reshape(V // 8, 8, D)
    NT = S // 8

    def gmap(j):
        return lambda i, idx_ref: (idx_ref[8 * i + j] >> 3, 0, 0)

    grid_spec = pltpu.PrefetchScalarGridSpec(
        num_scalar_prefetch=1,
        grid=(NT,),
        in_specs=[pl.BlockSpec((1, 8, D), gmap(j)) for j in range(8)],
        out_specs=pl.BlockSpec((1, 8, D), lambda i, idx_ref: (i, 0, 0)),
    )
    out = pl.pallas_call(
        _granule_body,
        grid_spec=grid_spec,
        out_shape=jax.ShapeDtypeStruct((NT, 8, D), jnp.float32),
        interpret=_INTERPRET,
    )(idx, *([emb3] * 8))
    return out.reshape(S, D)


def _gather_tc(emb, idx, S, D):
    V = emb.shape[0]
    emb3 = emb.reshape(V, 1, D)
    grid_spec = pltpu.PrefetchScalarGridSpec(
        num_scalar_prefetch=1,
        grid=(S,),
        in_specs=[pl.BlockSpec((1, 1, D), lambda i, idx_ref: (idx_ref[i], 0, 0))],
        out_specs=pl.BlockSpec((1, 1, D), lambda i, idx_ref: (i, 0, 0)),
    )
    out = pl.pallas_call(
        _gather_body,
        grid_spec=grid_spec,
        out_shape=jax.ShapeDtypeStruct((S, 1, D), jnp.float32),
        interpret=_INTERPRET,
    )(idx, emb3)
    return out.reshape(S, D)


def _gather_sc(emb, idx, S, D):
    info = plsc.get_sparse_core_info()
    NC, NS = info.num_cores, info.num_subcores
    NW = NC * NS
    b_per_w = S // NW
    mesh = plsc.VectorSubcoreMesh(core_axis_name="c", subcore_axis_name="s")

    @functools.partial(
        pl.kernel, mesh=mesh,
        out_type=jax.ShapeDtypeStruct((S, D), jnp.float32),
        scratch_types=[
            pltpu.VMEM((b_per_w,), jnp.int32),
            pltpu.VMEM((b_per_w, D), jnp.float32),
            pltpu.SemaphoreType.DMA,
        ],
        compiler_params=pltpu.CompilerParams(use_tc_tiling_on_sc=True),
    )
    def sc_gather(table_hbm, idx_hbm, out_hbm, idx_v, rows_v, sem):
        wid = lax.axis_index("s") * NC + lax.axis_index("c")
        base = wid * b_per_w
        pltpu.sync_copy(idx_hbm.at[pl.ds(base, b_per_w)], idx_v)
        pltpu.async_copy(table_hbm.at[idx_v], rows_v, sem).wait()
        pltpu.sync_copy(rows_v, out_hbm.at[pl.ds(base, b_per_w)])

    return sc_gather(emb, idx)


def _qkv_body(h_ref, s_ref, b_ref, w_ref, bqkv_ref, qkv_ref):
    hn = _ln_f32(h_ref[...], s_ref[...], b_ref[...])
    acc = jnp.dot(hn.astype(jnp.bfloat16), w_ref[...],
                  preferred_element_type=jnp.float32)
    qkv_ref[...] = (acc + bqkv_ref[...]).astype(jnp.bfloat16)


def _attn_body(q_ref, k_ref, v_ref, o_ref, *, BQ, S, H):
    i = pl.program_id(0)
    scale = 1.0 / np.sqrt(HD)
    rows = lax.broadcasted_iota(jnp.int32, (BQ, BQ), 0)
    cols = lax.broadcasted_iota(jnp.int32, (BQ, BQ), 1)
    diag_mask = rows >= cols
    for h in range(H):
        q = q_ref[:, h * HD:(h + 1) * HD]

        def body(j, carry):
            m, l, acc = carry
            kj = k_ref[pl.ds(j * BQ, BQ), h * HD:(h + 1) * HD]
            vj = v_ref[pl.ds(j * BQ, BQ), h * HD:(h + 1) * HD]
            sc = lax.dot_general(q, kj, (((1,), (1,)), ((), ())),
                                 preferred_element_type=jnp.float32) * scale
            sc = jnp.where(jnp.logical_or(j < i, diag_mask), sc, -1e9)
            m_new = jnp.maximum(m, jnp.max(sc, axis=-1, keepdims=True))
            alpha = jnp.exp(m - m_new)
            p = jnp.exp(sc - m_new)
            l = l * alpha + jnp.sum(p, axis=-1, keepdims=True)
            acc = acc * alpha + jnp.dot(p.astype(jnp.bfloat16), vj,
                                        preferred_element_type=jnp.float32)
            return m_new, l, acc

        m0 = jnp.full((BQ, 1), -1e30, jnp.float32)
        l0 = jnp.zeros((BQ, 1), jnp.float32)
        a0 = jnp.zeros((BQ, HD), jnp.float32)
        _, l, acc = lax.fori_loop(0, i + 1, body, (m0, l0, a0))
        o = acc * (1.0 / l)
        o_ref[:, h * HD:(h + 1) * HD] = o.astype(jnp.bfloat16)


def _post_body(o_ref, h_ref, wo_ref, bo_ref, s2_ref, b2ln_ref,
               w1_ref, b1_ref, w2_ref, b2_ref, out_ref, out_bf_ref):
    h = h_ref[...] + jnp.dot(o_ref[...], wo_ref[...],
                             preferred_element_type=jnp.float32) + bo_ref[...]
    hn2 = _ln_f32(h, s2_ref[...], b2ln_ref[...])
    ff = jnp.dot(hn2.astype(jnp.bfloat16), w1_ref[...],
                 preferred_element_type=jnp.float32) + b1_ref[...]
    ff = jax.nn.gelu(ff)
    out = h + jnp.dot(ff.astype(jnp.bfloat16), w2_ref[...],
                      preferred_element_type=jnp.float32) + b2_ref[...]
    out_ref[...] = out
    out_bf_ref[...] = out.astype(jnp.bfloat16)


def _head_body(h_ref, w_ref, o_ref):
    o_ref[...] = lax.dot_general(w_ref[...].astype(jnp.bfloat16), h_ref[...],
                                 (((1,), (1,)), ((), ())),
                                 preferred_element_type=jnp.float32)


def _layer(h, s1, b1ln, wqkv_bf, bqkv, wo_bf, bo, s2, b2ln,
           w1_bf, b1, w2_bf, b2, S, D, H, BS):
    NB = S // BS
    F = w1_bf.shape[1]
    qkv = pl.pallas_call(
        _qkv_body,
        grid=(NB,),
        in_specs=[
            pl.BlockSpec((BS, D), lambda i: (i, 0)),
            pl.BlockSpec((1, D), lambda i: (0, 0)),
            pl.BlockSpec((1, D), lambda i: (0, 0)),
            pl.BlockSpec((D, 3 * D), lambda i: (0, 0)),
            pl.BlockSpec((1, 3 * D), lambda i: (0, 0)),
        ],
        out_specs=pl.BlockSpec((BS, 3 * D), lambda i: (i, 0)),
        out_shape=jax.ShapeDtypeStruct((S, 3 * D), jnp.bfloat16),
        compiler_params=_PAR,
        interpret=_INTERPRET,
    )(h, s1, b1ln, wqkv_bf, bqkv)

    o = pl.pallas_call(
        functools.partial(_attn_body, BQ=BS, S=S, H=H),
        grid=(NB,),
        in_specs=[
            pl.BlockSpec((BS, D), lambda i: (i, 0)),
            pl.BlockSpec((S, D), lambda i: (0, 1)),
            pl.BlockSpec((S, D), lambda i: (0, 2)),
        ],
        out_specs=pl.BlockSpec((BS, D), lambda i: (i, 0)),
        out_shape=jax.ShapeDtypeStruct((S, D), jnp.bfloat16),
        compiler_params=_PAR,
        interpret=_INTERPRET,
    )(qkv, qkv, qkv)

    h = pl.pallas_call(
        _post_body,
        grid=(NB,),
        in_specs=[
            pl.BlockSpec((BS, D), lambda i: (i, 0)),
            pl.BlockSpec((BS, D), lambda i: (i, 0)),
            pl.BlockSpec((D, D), lambda i: (0, 0)),
            pl.BlockSpec((1, D), lambda i: (0, 0)),
            pl.BlockSpec((1, D), lambda i: (0, 0)),
            pl.BlockSpec((1, D), lambda i: (0, 0)),
            pl.BlockSpec((D, F), lambda i: (0, 0)),
            pl.BlockSpec((1, F), lambda i: (0, 0)),
            pl.BlockSpec((F, D), lambda i: (0, 0)),
            pl.BlockSpec((1, D), lambda i: (0, 0)),
        ],
        out_specs=[pl.BlockSpec((BS, D), lambda i: (i, 0)),
                   pl.BlockSpec((BS, D), lambda i: (i, 0))],
        out_shape=[jax.ShapeDtypeStruct((S, D), jnp.float32),
                   jax.ShapeDtypeStruct((S, D), jnp.bfloat16)],
        compiler_params=_PAR,
        interpret=_INTERPRET,
    )(o, h, wo_bf, bo, s2, b2ln, w1_bf, b1, w2_bf, b2)
    return h[0], h[1]


def kernel(x, emb, ln1_s, ln1_b, wqkv, bqkv, wo, bo, ln2_s, ln2_b,
           w1, b1, w2, b2, w_out):
    B, S = x.shape
    V, D = emb.shape
    L = wqkv.shape[0]
    H = D // HD
    BS = 256
    VB = 1000

    idx = x.reshape(S).astype(jnp.int32)
    h = _gather_sc(emb, idx, S, D)

    bf = jnp.bfloat16
    h_bf = None
    for l in range(L):
        h, h_bf = _layer(
            h,
            ln1_s[l].reshape(1, D), ln1_b[l].reshape(1, D),
            wqkv[l].astype(bf), bqkv[l].reshape(1, 3 * D),
            wo[l].astype(bf), bo[l].reshape(1, D),
            ln2_s[l].reshape(1, D), ln2_b[l].reshape(1, D),
            w1[l].astype(bf), b1[l].reshape(1, -1),
            w2[l].astype(bf), b2[l].reshape(1, D),
            S, D, H, BS,
        )

    NV = (V + VB - 1) // VB
    logits_t = pl.pallas_call(
        _head_body,
        grid=(NV,),
        in_specs=[
            pl.BlockSpec((S, D), lambda j: (0, 0)),
            pl.BlockSpec((VB, D), lambda j: (j, 0)),
        ],
        out_specs=pl.BlockSpec((VB, S), lambda j: (j, 0)),
        out_shape=jax.ShapeDtypeStruct((V, S), jnp.float32),
        compiler_params=_PAR,
        interpret=_INTERPRET,
    )(h_bf, w_out)
    return logits_t.T.reshape(B, S, V)
